# baseline (speedup 1.0000x reference)
import functools

import jax
import jax.numpy as jnp
import numpy as np
from jax import lax
from jax.experimental import pallas as pl
from jax.experimental.pallas import tpu as pltpu
from jax.experimental.pallas import tpu_sc as plsc

N = 10000
E = 320000
D = 128
H = 4
C = 128
HC = H * C
KV = 2 * HC
NW = 32
CH = 48
EPAD = ((E + CH - 1) // CH) * CH
NEG = -3.0e38
_ROWS_BLK = 1000
_ISCALE = float(1.0 / np.sqrt(C))
_INTERP = False


def _mm3_body(s_w, x_ref, w_ref, b_ref, q_ref, kv_ref, s_ref):
    y = jnp.dot(x_ref[...], w_ref[...], preferred_element_type=jnp.float32)
    y = y + b_ref[...]
    q_ref[...] = y[:, :HC]
    kv_ref[...] = y[:, HC:HC + KV]
    s_ref[...] = y[:, HC + KV:]


def _proj(x, w, b, s_w):
    n, d = x.shape
    m = w.shape[1]
    rb = min(n, _ROWS_BLK)
    grid = (n // rb,)
    return pl.pallas_call(
        functools.partial(_mm3_body, s_w),
        grid=grid,
        interpret=_INTERP,
        in_specs=[
            pl.BlockSpec((rb, d), lambda i: (i, 0)),
            pl.BlockSpec((d, m), lambda i: (0, 0)),
            pl.BlockSpec((1, m), lambda i: (0, 0)),
        ],
        out_specs=[
            pl.BlockSpec((rb, HC), lambda i: (i, 0)),
            pl.BlockSpec((rb, KV), lambda i: (i, 0)),
            pl.BlockSpec((rb, s_w), lambda i: (i, 0)),
        ],
        out_shape=[
            jax.ShapeDtypeStruct((n, HC), jnp.float32),
            jax.ShapeDtypeStruct((n, KV), jnp.float32),
            jax.ShapeDtypeStruct((n, s_w), jnp.float32),
        ],
    )(x, w, b.reshape(1, m))


_GDN = lax.GatherDimensionNumbers(
    offset_dims=(), collapsed_slice_dims=(0,), start_index_map=(0,))


def _gather16(v, idx):
    return lax.gather(v, idx[:, None], dimension_numbers=_GDN,
                      slice_sizes=(1,),
                      mode=lax.GatherScatterMode.PROMISE_IN_BOUNDS)


def _allsum(v):
    iot = lax.iota(jnp.int32, 16)
    for k in (1, 2, 4, 8):
        v = v + _gather16(v, iot ^ k)
    return v


def _bcast_lane(v, h):
    return _gather16(v, jnp.full((16,), h, jnp.int32))


def _sread(ref, i):
    return ref[pl.ds(i, 16)][0]


def _sc_attn_call(q, kv, s_res, sd, estart, nstart, *, out_w, concat):
    mesh = plsc.VectorSubcoreMesh(core_axis_name="c", subcore_axis_name="s",
                                  num_cores=2, num_subcores=16)

    @functools.partial(
        pl.kernel,
        mesh=mesh,
        interpret=_INTERP,
        out_type=jax.ShapeDtypeStruct((N, out_w), jnp.float32),
        scratch_types=[
            pltpu.VMEM((48,), jnp.int32),
            pltpu.VMEM((48,), jnp.int32),
            pltpu.VMEM((2, 2, CH + 16), jnp.int32),
            pltpu.VMEM((2, CH, KV), jnp.float32),
            pltpu.VMEM((1, HC), jnp.float32),
            pltpu.VMEM((1, out_w), jnp.float32),
            pltpu.VMEM((HC,), jnp.float32),
            pltpu.VMEM((1, out_w), jnp.float32),
            pltpu.SemaphoreType.DMA,
            pltpu.SemaphoreType.DMA,
        ],
    )
    def body(q_hbm, kv_hbm, s_hbm, sd_hbm, estart_hbm,
             nstart_hbm, out_hbm,
             ebuf, nbuf, sdb, kvb, qb, sb, accb, ob, sem0, sem1):
        w = lax.axis_index("s") * 2 + lax.axis_index("c")
        pltpu.sync_copy(estart_hbm, ebuf)
        pltpu.sync_copy(nstart_hbm, nbuf)
        e0 = _sread(ebuf, w)
        e1 = _sread(ebuf, w + 1)
        n_lo = _sread(nbuf, w)
        n_hi = _sread(nbuf, w + 1)

        def gstart(c, b):
            def do(bb):
                pltpu.sync_copy(sd_hbm.at[c], sdb.at[bb])
                pltpu.make_async_copy(
                    kv_hbm.at[sdb.at[bb, 0, pl.ds(0, CH)]],
                    kvb.at[bb], sem0 if bb == 0 else sem1).start()

            lax.cond(b == 0, lambda _: (do(0), 0)[1],
                     lambda _: (do(1), 0)[1], 0)

        def gwait(b):
            def do(bb):
                pltpu.make_async_copy(
                    kv_hbm.at[sdb.at[bb, 0, pl.ds(0, CH)]],
                    kvb.at[bb], sem0 if bb == 0 else sem1).wait()

            lax.cond(b == 0, lambda _: (do(0), 0)[1],
                     lambda _: (do(1), 0)[1], 0)

        iot = lax.iota(jnp.int32, 16)
        zero16 = jnp.zeros((16,), jnp.float32)
        neg16 = jnp.full((16,), NEG, jnp.float32)

        def zero_acc():
            for i in range(HC // 16):
                accb[pl.ds(i * 16, 16)] = zero16

        def write_node(d_node, s_vec):
            rs = 1.0 / s_vec
            if concat:
                for h in range(H):
                    rh = rs[h]
                    for i in range(C // 16):
                        o = accb[pl.ds(h * C + i * 16, 16)] * rh
                        o = o + sb[0, pl.ds(h * C + i * 16, 16)]
                        ob[0, pl.ds(h * C + i * 16, 16)] = jnp.maximum(o, 0.0)
            else:
                rh = [rs[h] * 0.25 for h in range(H)]
                for i in range(C // 16):
                    o = sb[0, pl.ds(i * 16, 16)]
                    for h in range(H):
                        o = o + accb[pl.ds(h * C + i * 16, 16)] * rh[h]
                    ob[0, pl.ds(i * 16, 16)] = o
            pltpu.sync_copy(ob, out_hbm.at[pl.ds(d_node, 1)])

        def start_node(d_node):
            pltpu.sync_copy(s_hbm.at[pl.ds(d_node, 1)], sb)
            pltpu.sync_copy(q_hbm.at[pl.ds(d_node, 1)], qb)
            zero_acc()

        def gap_fill(lo, hi):
            def gbody(n, _):
                pltpu.sync_copy(s_hbm.at[pl.ds(n, 1)], sb)
                if concat:
                    for i in range(out_w // 16):
                        ob[0, pl.ds(i * 16, 16)] = jnp.maximum(
                            sb[0, pl.ds(i * 16, 16)], 0.0)
                    pltpu.sync_copy(ob, out_hbm.at[pl.ds(n, 1)])
                else:
                    pltpu.sync_copy(sb, out_hbm.at[pl.ds(n, 1)])
                return 0
            lax.fori_loop(lo, hi, gbody, 0)

        def edge_body(j, carry, b):
            d_prev, m_vec, s_vec = carry
            d = sdb[b, 1, pl.ds(j, 16)][0]
            is_new = d != d_prev

            def on_new(_):
                lax.cond(d_prev >= 0,
                         lambda _: (write_node(d_prev, s_vec), 0)[1],
                         lambda _: 0, 0)
                gap_fill(jnp.where(d_prev >= 0, d_prev + 1, n_lo), d)
                start_node(d)
                return 0

            lax.cond(is_new, on_new, lambda _: 0, 0)
            m_vec = jnp.where(is_new, neg16, m_vec)
            s_vec = jnp.where(is_new, zero16, s_vec)

            alpha = zero16
            for h in range(H):
                t = zero16
                for i in range(C // 16):
                    t = t + (qb[0, pl.ds(h * C + i * 16, 16)]
                             * kvb[b, j, pl.ds(h * C + i * 16, 16)])
                alpha = jnp.where(iot == h, _allsum(t)[0] * _ISCALE, alpha)

            m_new = jnp.maximum(m_vec, alpha)
            scale = jnp.exp(m_vec - m_new)
            ex = jnp.exp(alpha - m_new)
            s_new = s_vec * scale + ex

            for h in range(H):
                sc_h = scale[h]
                ex_h = ex[h]
                for i in range(C // 16):
                    a = accb[pl.ds(h * C + i * 16, 16)]
                    a = (a * sc_h
                         + ex_h * kvb[b, j, pl.ds(HC + h * C + i * 16, 16)])
                    accb[pl.ds(h * C + i * 16, 16)] = a
            return d, m_new, s_new

        c0 = e0 // CH
        c1 = (e1 + CH - 1) // CH

        def chunk_body(c, carry):
            ce = c * CH
            b = lax.rem(c - c0, 2)
            lax.cond(c + 1 < c1,
                     lambda _: (gstart(c + 1, 1 - b), 0)[1],
                     lambda _: 0, 0)
            gwait(b)
            lo = jnp.maximum(ce, e0) - ce
            hi = jnp.minimum(ce + CH, e1) - ce
            return lax.fori_loop(lo, hi,
                                 functools.partial(edge_body, b=b), carry)

        init = (jnp.int32(-1), neg16, zero16)
        lax.cond(c0 < c1, lambda _: (gstart(c0, 0), 0)[1], lambda _: 0, 0)
        d_prev, m_vec, s_vec = lax.fori_loop(c0, c1, chunk_body, init)
        lax.cond(d_prev >= 0,
                 lambda _: (write_node(d_prev, s_vec), 0)[1],
                 lambda _: 0, 0)
        gap_fill(jnp.where(d_prev >= 0, d_prev + 1, n_lo), n_hi)

    return body(q, kv, s_res, sd, estart, nstart)


def kernel(x, edge_index, Wq1, bq1, Wk1, bk1, Wv1, bv1, Ws1, bs1,
           Wq2, bq2, Wk2, bk2, Wv2, bv2, Ws2, bs2):
    src = edge_index[0].astype(jnp.int32)
    dst = edge_index[1].astype(jnp.int32)

    order = jnp.argsort(dst).astype(jnp.int32)
    dsts = dst[order]
    srcs = src[order]
    zpad = jnp.zeros((EPAD + 16 - E,), jnp.int32)
    win = (jnp.arange(EPAD // CH, dtype=jnp.int32)[:, None] * CH
           + jnp.arange(CH + 16, dtype=jnp.int32)[None, :])
    sd = jnp.stack([jnp.concatenate([srcs, zpad])[win],
                    jnp.concatenate([dsts, zpad])[win]], axis=1)
    split = (jnp.arange(1, NW, dtype=jnp.int32) * (E // NW))
    bnodes = dsts[split]
    inner = jnp.searchsorted(dsts, bnodes, side="left").astype(jnp.int32)
    pad = jnp.zeros((15,), jnp.int32)
    estart = jnp.concatenate([
        jnp.zeros((1,), jnp.int32), inner,
        jnp.full((1,), E, jnp.int32), pad])
    nstart = jnp.concatenate([
        jnp.zeros((1,), jnp.int32), bnodes,
        jnp.full((1,), N, jnp.int32), pad])

    W1 = jnp.concatenate([Wq1, Wk1, Wv1, Ws1], axis=1)
    b1 = jnp.concatenate([bq1, bk1, bv1, bs1], axis=0)
    q1, kv1, s1 = _proj(x, W1, b1, HC)
    h = _sc_attn_call(q1, kv1, s1, sd, estart, nstart,
                      out_w=HC, concat=True)

    W2 = jnp.concatenate([Wq2, Wk2, Wv2, Ws2], axis=1)
    b2 = jnp.concatenate([bq2, bk2, bv2, bs2], axis=0)
    q2, kv2, s2 = _proj(h, W2, b2, C)
    return _sc_attn_call(q2, kv2, s2, sd, estart, nstart,
                         out_w=C, concat=False)


# speedup vs baseline: 1.0295x; 1.0295x over previous
import functools

import jax
import jax.numpy as jnp
import numpy as np
from jax import lax
from jax.experimental import pallas as pl
from jax.experimental.pallas import tpu as pltpu
from jax.experimental.pallas import tpu_sc as plsc

N = 10000
E = 320000
D = 128
H = 4
C = 128
HC = H * C
KV = 2 * HC
NW = 32
CH = 48
EPAD = ((E + CH - 1) // CH) * CH
NEG = -3.0e38
_ROWS_BLK = 1000
_ISCALE = float(1.0 / np.sqrt(C))


def _mm3_body(s_w, x_ref, w_ref, b_ref, q_ref, kv_ref, s_ref):
    y = jnp.dot(x_ref[...], w_ref[...], preferred_element_type=jnp.float32)
    y = y + b_ref[...]
    q_ref[...] = y[:, :HC]
    kv_ref[...] = y[:, HC:HC + KV]
    s_ref[...] = y[:, HC + KV:]


def _proj(x, w, b, s_w):
    n, d = x.shape
    m = w.shape[1]
    rb = min(n, _ROWS_BLK)
    grid = (n // rb,)
    return pl.pallas_call(
        functools.partial(_mm3_body, s_w),
        grid=grid,
        in_specs=[
            pl.BlockSpec((rb, d), lambda i: (i, 0)),
            pl.BlockSpec((d, m), lambda i: (0, 0)),
            pl.BlockSpec((1, m), lambda i: (0, 0)),
        ],
        out_specs=[
            pl.BlockSpec((rb, HC), lambda i: (i, 0)),
            pl.BlockSpec((rb, KV), lambda i: (i, 0)),
            pl.BlockSpec((rb, s_w), lambda i: (i, 0)),
        ],
        out_shape=[
            jax.ShapeDtypeStruct((n, HC), jnp.float32),
            jax.ShapeDtypeStruct((n, KV), jnp.float32),
            jax.ShapeDtypeStruct((n, s_w), jnp.float32),
        ],
    )(x, w, b.reshape(1, m))


_GDN = lax.GatherDimensionNumbers(
    offset_dims=(), collapsed_slice_dims=(0,), start_index_map=(0,))


def _gather16(v, idx):
    return lax.gather(v, idx[:, None], dimension_numbers=_GDN,
                      slice_sizes=(1,),
                      mode=lax.GatherScatterMode.PROMISE_IN_BOUNDS)


def _allsum(v):
    iot = lax.iota(jnp.int32, 16)
    for k in (1, 2, 4, 8):
        v = v + _gather16(v, iot ^ k)
    return v


def _bcast_lane(v, h):
    return _gather16(v, jnp.full((16,), h, jnp.int32))


def _sread(ref, i):
    return ref[pl.ds(i, 16)][0]


def _sc_attn_call(q, kv, s_res, sd, estart, nstart, *, out_w, concat):
    mesh = plsc.VectorSubcoreMesh(core_axis_name="c", subcore_axis_name="s",
                                  num_cores=2, num_subcores=16)

    @functools.partial(
        pl.kernel,
        mesh=mesh,
        out_type=jax.ShapeDtypeStruct((N, out_w), jnp.float32),
        scratch_types=[
            pltpu.VMEM((48,), jnp.int32),
            pltpu.VMEM((48,), jnp.int32),
            pltpu.VMEM((2, 2, CH + 16), jnp.int32),
            pltpu.VMEM((2, CH, KV), jnp.float32),
            pltpu.VMEM((1, HC), jnp.float32),
            pltpu.VMEM((1, out_w), jnp.float32),
            pltpu.VMEM((HC,), jnp.float32),
            pltpu.VMEM((1, out_w), jnp.float32),
            pltpu.SemaphoreType.DMA,
            pltpu.SemaphoreType.DMA,
        ],
    )
    def body(q_hbm, kv_hbm, s_hbm, sd_hbm, estart_hbm,
             nstart_hbm, out_hbm,
             ebuf, nbuf, sdb, kvb, qb, sb, accb, ob, sem0, sem1):
        w = lax.axis_index("s") * 2 + lax.axis_index("c")
        pltpu.sync_copy(estart_hbm, ebuf)
        pltpu.sync_copy(nstart_hbm, nbuf)
        e0 = _sread(ebuf, w)
        e1 = _sread(ebuf, w + 1)
        n_lo = _sread(nbuf, w)
        n_hi = _sread(nbuf, w + 1)

        def gstart(c, b):
            def do(bb):
                pltpu.sync_copy(sd_hbm.at[c], sdb.at[bb])
                pltpu.make_async_copy(
                    kv_hbm.at[sdb.at[bb, 0, pl.ds(0, CH)]],
                    kvb.at[bb], sem0 if bb == 0 else sem1).start()

            lax.cond(b == 0, lambda _: (do(0), 0)[1],
                     lambda _: (do(1), 0)[1], 0)

        def gwait(b):
            def do(bb):
                pltpu.make_async_copy(
                    kv_hbm.at[sdb.at[bb, 0, pl.ds(0, CH)]],
                    kvb.at[bb], sem0 if bb == 0 else sem1).wait()

            lax.cond(b == 0, lambda _: (do(0), 0)[1],
                     lambda _: (do(1), 0)[1], 0)

        iot = lax.iota(jnp.int32, 16)
        zero16 = jnp.zeros((16,), jnp.float32)
        neg16 = jnp.full((16,), NEG, jnp.float32)

        def zero_acc():
            for i in range(HC // 16):
                accb[pl.ds(i * 16, 16)] = zero16

        def write_node(d_node, s_vec):
            rs = 1.0 / s_vec
            if concat:
                for h in range(H):
                    rh = _bcast_lane(rs, h)
                    for i in range(C // 16):
                        o = accb[pl.ds(h * C + i * 16, 16)] * rh
                        o = o + sb[0, pl.ds(h * C + i * 16, 16)]
                        ob[0, pl.ds(h * C + i * 16, 16)] = jnp.maximum(o, 0.0)
            else:
                rh = [_bcast_lane(rs, h) * 0.25 for h in range(H)]
                for i in range(C // 16):
                    o = sb[0, pl.ds(i * 16, 16)]
                    for h in range(H):
                        o = o + accb[pl.ds(h * C + i * 16, 16)] * rh[h]
                    ob[0, pl.ds(i * 16, 16)] = o
            pltpu.sync_copy(ob, out_hbm.at[pl.ds(d_node, 1)])

        def start_node(d_node):
            pltpu.sync_copy(s_hbm.at[pl.ds(d_node, 1)], sb)
            pltpu.sync_copy(q_hbm.at[pl.ds(d_node, 1)], qb)
            zero_acc()

        def gap_fill(lo, hi):
            def gbody(n, _):
                pltpu.sync_copy(s_hbm.at[pl.ds(n, 1)], sb)
                if concat:
                    for i in range(out_w // 16):
                        ob[0, pl.ds(i * 16, 16)] = jnp.maximum(
                            sb[0, pl.ds(i * 16, 16)], 0.0)
                    pltpu.sync_copy(ob, out_hbm.at[pl.ds(n, 1)])
                else:
                    pltpu.sync_copy(sb, out_hbm.at[pl.ds(n, 1)])
                return 0
            lax.fori_loop(lo, hi, gbody, 0)

        def edge_body(j, carry, b):
            d_prev, m_vec, s_vec = carry
            d = sdb[b, 1, pl.ds(j, 16)][0]
            is_new = d != d_prev

            def on_new(_):
                lax.cond(d_prev >= 0,
                         lambda _: (write_node(d_prev, s_vec), 0)[1],
                         lambda _: 0, 0)
                gap_fill(jnp.where(d_prev >= 0, d_prev + 1, n_lo), d)
                start_node(d)
                return 0

            lax.cond(is_new, on_new, lambda _: 0, 0)
            m_vec = jnp.where(is_new, neg16, m_vec)
            s_vec = jnp.where(is_new, zero16, s_vec)

            alpha = zero16
            for h in range(H):
                t = zero16
                for i in range(C // 16):
                    t = t + (qb[0, pl.ds(h * C + i * 16, 16)]
                             * kvb[b, j, pl.ds(h * C + i * 16, 16)])
                alpha = jnp.where(iot == h, _allsum(t) * _ISCALE, alpha)

            m_new = jnp.maximum(m_vec, alpha)
            scale = jnp.exp(m_vec - m_new)
            ex = jnp.exp(alpha - m_new)
            s_new = s_vec * scale + ex

            for h in range(H):
                sc_h = _bcast_lane(scale, h)
                ex_h = _bcast_lane(ex, h)
                for i in range(C // 16):
                    a = accb[pl.ds(h * C + i * 16, 16)]
                    a = (a * sc_h
                         + ex_h * kvb[b, j, pl.ds(HC + h * C + i * 16, 16)])
                    accb[pl.ds(h * C + i * 16, 16)] = a
            return d, m_new, s_new

        c0 = e0 // CH
        c1 = (e1 + CH - 1) // CH

        def chunk_body(c, carry):
            ce = c * CH
            b = lax.rem(c - c0, 2)
            lax.cond(c + 1 < c1,
                     lambda _: (gstart(c + 1, 1 - b), 0)[1],
                     lambda _: 0, 0)
            gwait(b)
            lo = jnp.maximum(ce, e0) - ce
            hi = jnp.minimum(ce + CH, e1) - ce
            return lax.fori_loop(lo, hi,
                                 functools.partial(edge_body, b=b), carry)

        init = (jnp.int32(-1), neg16, zero16)
        lax.cond(c0 < c1, lambda _: (gstart(c0, 0), 0)[1], lambda _: 0, 0)
        d_prev, m_vec, s_vec = lax.fori_loop(c0, c1, chunk_body, init)
        lax.cond(d_prev >= 0,
                 lambda _: (write_node(d_prev, s_vec), 0)[1],
                 lambda _: 0, 0)
        gap_fill(jnp.where(d_prev >= 0, d_prev + 1, n_lo), n_hi)

    return body(q, kv, s_res, sd, estart, nstart)


def kernel(x, edge_index, Wq1, bq1, Wk1, bk1, Wv1, bv1, Ws1, bs1,
           Wq2, bq2, Wk2, bk2, Wv2, bv2, Ws2, bs2):
    src = edge_index[0].astype(jnp.int32)
    dst = edge_index[1].astype(jnp.int32)

    order = jnp.argsort(dst).astype(jnp.int32)
    dsts = dst[order]
    srcs = src[order]
    zpad = jnp.zeros((EPAD + 16 - E,), jnp.int32)
    win = (jnp.arange(EPAD // CH, dtype=jnp.int32)[:, None] * CH
           + jnp.arange(CH + 16, dtype=jnp.int32)[None, :])
    sd = jnp.stack([jnp.concatenate([srcs, zpad])[win],
                    jnp.concatenate([dsts, zpad])[win]], axis=1)
    split = (jnp.arange(1, NW, dtype=jnp.int32) * (E // NW))
    bnodes = dsts[split]
    inner = jnp.searchsorted(dsts, bnodes, side="left").astype(jnp.int32)
    pad = jnp.zeros((15,), jnp.int32)
    estart = jnp.concatenate([
        jnp.zeros((1,), jnp.int32), inner,
        jnp.full((1,), E, jnp.int32), pad])
    nstart = jnp.concatenate([
        jnp.zeros((1,), jnp.int32), bnodes,
        jnp.full((1,), N, jnp.int32), pad])

    W1 = jnp.concatenate([Wq1, Wk1, Wv1, Ws1], axis=1)
    b1 = jnp.concatenate([bq1, bk1, bv1, bs1], axis=0)
    q1, kv1, s1 = _proj(x, W1, b1, HC)
    h = _sc_attn_call(q1, kv1, s1, sd, estart, nstart,
                      out_w=HC, concat=True)

    W2 = jnp.concatenate([Wq2, Wk2, Wv2, Ws2], axis=1)
    b2 = jnp.concatenate([bq2, bk2, bv2, bs2], axis=0)
    q2, kv2, s2 = _proj(h, W2, b2, C)
    return _sc_attn_call(q2, kv2, s2, sd, estart, nstart,
                         out_w=C, concat=False)
